# baseline (speedup 1.0000x reference)
import functools

import jax
import jax.numpy as jnp
from jax import lax
from jax.experimental import pallas as pl
from jax.experimental.pallas import tpu as pltpu
from jax.experimental.pallas import tpu_sc as plsc

N = 10000
E = 320000
D = 128
H = 4
EPS = 1e-5

NC = 2
NS = 16
NW = NC * NS
EW = E // NW
NP = 10240
ROWS = NP // NS


_K = 40
_NCH = EW // _K
_ZR = 64
_CW = 128


def _sc_mesh():
    return plsc.VectorSubcoreMesh(
        core_axis_name="c", subcore_axis_name="s",
        num_cores=NC, num_subcores=NS)


def _make_sc_agg():
    K, NCH = _K, _NCH

    @functools.partial(
        pl.kernel,
        mesh=_sc_mesh(),
        out_type=jax.ShapeDtypeStruct((NC, NP, D), jnp.float32),
        scratch_types=[
            pltpu.VMEM_SHARED((NP, D), jnp.float32),
            pltpu.VMEM((EW,), jnp.int32),
            pltpu.VMEM((2, K), jnp.int32),
            pltpu.VMEM((2, K, D), jnp.float32),
            pltpu.VMEM((2, K, D), jnp.float32),
            pltpu.VMEM((2, K, D), jnp.float32),
            pltpu.SemaphoreType.DMA,
            pltpu.SemaphoreType.DMA,
            pltpu.SemaphoreType.DMA,
            pltpu.SemaphoreType.DMA,
            pltpu.SemaphoreType.DMA,
            pltpu.SemaphoreType.DMA,
            pltpu.SemaphoreType.DMA,
            pltpu.SemaphoreType.DMA,
        ],
    )
    def sc_agg(xl_hbm, att_hbm, src_hbm, dst_hbm, out_hbm,
               acc, src_t, dstv, xj, attv, prod,
               semg0, semg1, sema0, sema1, semd0, semd1, sems0, sems1):
        semg = (semg0, semg1)
        sema = (sema0, sema1)
        semd = (semd0, semd1)
        sems = (sems0, sems1)
        c = lax.axis_index("c")
        s = lax.axis_index("s")
        w = s * NC + c
        ebase = w * EW
        row0 = s * ROWS

        def zrow(r, carry):
            for j in range(D // 16):
                xj[0, r, pl.ds(j * 16, 16)] = jnp.zeros((16,), jnp.float32)
            return carry
        lax.fori_loop(0, K, zrow, 0)
        for k5 in range(ROWS // K):
            pltpu.sync_copy(xj.at[0], acc.at[pl.ds(row0 + k5 * K, K)])

        pltpu.sync_copy(src_hbm.at[pl.ds(ebase, EW)], src_t)

        plsc.subcore_barrier()

        def issue_gather(ci, b):
            pltpu.async_copy(
                xl_hbm.at[src_t.at[pl.ds(ci * K, K)]], xj.at[b], semg[b])

        def issue_att(ci, b):
            pltpu.async_copy(att_hbm.at[pl.ds(ebase + ci * K, K)],
                             attv.at[b], sema[b])

        def issue_dst(ci, b):
            pltpu.async_copy(dst_hbm.at[pl.ds(ebase + ci * K, K)],
                             dstv.at[b], semd[b])

        def drain_scatter(b):
            pltpu.make_async_copy(prod.at[b], acc.at[dstv.at[b]],
                                  sems[b]).wait()

        def step(ci, b):
            pltpu.make_async_copy(
                xl_hbm.at[src_t.at[pl.ds(ci * K, K)]], xj.at[b],
                semg[b]).wait()
            pltpu.make_async_copy(att_hbm.at[pl.ds(ebase + ci * K, K)],
                                  attv.at[b], sema[b]).wait()

            @pl.when(ci >= 2)
            def _():
                drain_scatter(b)
            issue_dst(ci, b)

            def mrow(r, carry2):
                for j in range(D // 16):
                    sl = pl.ds(j * 16, 16)
                    prod[b, r, sl] = xj[b, r, sl] * attv[b, r, sl]
                return carry2
            lax.fori_loop(0, K, mrow, 0)

            @pl.when(ci + 2 < NCH)
            def _():
                issue_gather(ci + 2, b)
                issue_att(ci + 2, b)

            pltpu.make_async_copy(dst_hbm.at[pl.ds(ebase + ci * K, K)],
                                  dstv.at[b], semd[b]).wait()
            pltpu.async_copy(prod.at[b], acc.at[dstv.at[b]], sems[b],
                             add=True)

        issue_gather(0, 0)
        issue_att(0, 0)
        issue_gather(1, 1)
        issue_att(1, 1)

        def pair(i, carry):
            step(2 * i, 0)
            step(2 * i + 1, 1)
            return carry
        lax.fori_loop(0, NCH // 2, pair, 0)

        drain_scatter(0)
        drain_scatter(1)

        plsc.subcore_barrier()
        pltpu.sync_copy(acc.at[pl.ds(row0, ROWS)],
                        out_hbm.at[c].at[pl.ds(row0, ROWS)])

    return sc_agg


_KC = 80
_NCHC = EW // _KC


def _make_sc_counts():
    K, NCH, ZR, CW = _KC, _NCHC, _ZR, _CW

    @functools.partial(
        pl.kernel,
        mesh=_sc_mesh(),
        out_type=jax.ShapeDtypeStruct((NC, NP, CW), jnp.float32),
        scratch_types=[
            pltpu.VMEM_SHARED((NP, CW), jnp.float32),
            pltpu.VMEM((NCH, K), jnp.int32),
            pltpu.VMEM((K, CW), jnp.float32),
            pltpu.VMEM((ZR, CW), jnp.float32),
            pltpu.SemaphoreType.DMA,
            pltpu.SemaphoreType.DMA,
        ],
    )
    def sc_counts(dst_hbm, cnt_hbm, cacc, dst_t, onesb, czbuf, sem0, sem1):
        sems = (sem0, sem1)
        c = lax.axis_index("c")
        s = lax.axis_index("s")
        w = s * NC + c

        def zrow(r, carry):
            czbuf[r, :] = jnp.zeros((CW,), jnp.float32)
            return carry
        lax.fori_loop(0, ZR, zrow, 0)
        row0 = s * ROWS
        for k5 in range(ROWS // ZR):
            pltpu.sync_copy(czbuf, cacc.at[pl.ds(row0 + k5 * ZR, ZR)])

        def orow(r, carry):
            onesb[r, :] = jnp.ones((CW,), jnp.float32)
            return carry
        lax.fori_loop(0, K, orow, 0)

        pltpu.sync_copy(dst_hbm.at[w], dst_t)

        plsc.subcore_barrier()

        def issue(ci, b):
            pltpu.async_copy(onesb, cacc.at[dst_t.at[ci]], sems[b],
                             add=True)

        def drain(b):
            pltpu.make_async_copy(onesb, cacc.at[dst_t.at[0]],
                                  sems[b]).wait()

        issue(0, 0)
        issue(1, 1)

        def pair(i, carry):
            drain(0)

            @pl.when(2 * i + 2 < NCH)
            def _():
                issue(2 * i + 2, 0)
            drain(1)

            @pl.when(2 * i + 3 < NCH)
            def _():
                issue(2 * i + 3, 1)
            return carry
        lax.fori_loop(0, NCH // 2, pair, 0)
        drain(0)

        plsc.subcore_barrier()
        pltpu.sync_copy(cacc.at[pl.ds(row0, ROWS)],
                        cnt_hbm.at[c].at[pl.ds(row0, ROWS)])

    return sc_counts


@functools.lru_cache(maxsize=None)
def _get_sc_agg():
    return _make_sc_agg()


@functools.lru_cache(maxsize=None)
def _get_sc_counts():
    return _make_sc_counts()


_EB = 4000


def _att_layer(ea, g_mat, w1, b1, w2, b2, aout):
    h = jnp.maximum(
        jnp.dot(ea, w1[...], preferred_element_type=jnp.float32) + b1[...],
        0.0)
    ew = jnp.dot(h, w2[...], preferred_element_type=jnp.float32) + b2[...]
    e = jnp.exp(ew)
    denom = jnp.dot(e, g_mat, preferred_element_type=jnp.float32)
    aout[...] = e / denom


def _att0_body(ea_ref, g_ref, w10, b10, w20, b20, a0_ref):
    _att_layer(ea_ref[...], g_ref[...], w10, b10, w20, b20, a0_ref)


def _att12_body(ea_ref, g_ref, w11, b11, w21, b21, w12, b12, w22, b22,
                a1_ref, a2_ref, eout_ref):
    ea = ea_ref[...]
    _att_layer(ea, g_ref[...], w11, b11, w21, b21, a1_ref)
    _att_layer(ea, g_ref[...], w12, b12, w22, b22, a2_ref)
    eout_ref[...] = ea


def _head_mask():
    i = jnp.arange(D)
    return (i[:, None] // (D // H) == i[None, :] // (D // H)).astype(jnp.float32)


def _att0(edge_attr, g_mat, ws):
    mat = pl.BlockSpec((D, D), lambda i: (0, 0))
    vec = pl.BlockSpec((1, D), lambda i: (0, 0))
    blk = pl.BlockSpec((_EB, D), lambda i: (i, 0))
    return pl.pallas_call(
        _att0_body,
        grid=(E // _EB,),
        in_specs=[blk, mat, mat, vec, mat, vec],
        out_specs=blk,
        out_shape=jax.ShapeDtypeStruct((E, D), jnp.float32),
    )(edge_attr, g_mat, *ws)


def _att12(edge_attr, g_mat, ws):
    mat = pl.BlockSpec((D, D), lambda i: (0, 0))
    vec = pl.BlockSpec((1, D), lambda i: (0, 0))
    blk = pl.BlockSpec((_EB, D), lambda i: (i, 0))
    return pl.pallas_call(
        _att12_body,
        grid=(E // _EB,),
        in_specs=[blk, mat] + [mat, vec, mat, vec] * 2,
        out_specs=[blk, blk, blk],
        out_shape=[jax.ShapeDtypeStruct((E, D), jnp.float32)] * 3,
    )(edge_attr, g_mat, *ws)


def _lin_body(x_ref, w_ref, b_ref, o_ref):
    o_ref[...] = (jnp.dot(x_ref[...], w_ref[...],
                          preferred_element_type=jnp.float32) + b_ref[...])


def _lin(x, w, b):
    return pl.pallas_call(
        _lin_body,
        out_shape=jax.ShapeDtypeStruct((N, D), jnp.float32),
    )(x, w, b)


def _bnorm(h, g, b):
    m = jnp.mean(h, axis=0, keepdims=True)
    v = jnp.mean((h - m) * (h - m), axis=0, keepdims=True)
    return (h - m) / jnp.sqrt(v + EPS) * g + b


def _comb0_body(p_ref, c_ref, g_ref, bt_ref, w_ref, b_ref,
                h0_ref, xl1_ref, cinv_ref):
    su = p_ref[0, :N, :D] + p_ref[1, :N, :D]
    cnt = c_ref[0, :N] + c_ref[1, :N]
    cinv = 1.0 / jnp.maximum(cnt[:, 0:1], 1.0)
    h = su * cinv
    h0 = jnp.maximum(_bnorm(h, g_ref[...], bt_ref[...]), 0.0)
    h0_ref[...] = h0
    xl1_ref[...] = (jnp.dot(h0, w_ref[...],
                            preferred_element_type=jnp.float32) + b_ref[...])
    cinv_ref[...] = jnp.broadcast_to(cinv, (N, D))


def _comb0(p, cnt, g, bt, w, b):
    return pl.pallas_call(
        _comb0_body,
        out_shape=[jax.ShapeDtypeStruct((N, D), jnp.float32)] * 3,
    )(p, cnt, g, bt, w, b)


def _comb1_body(p_ref, cinv_ref, g_ref, bt_ref, w_ref, b_ref, xl2_ref):
    h = (p_ref[0, :N] + p_ref[1, :N]) * cinv_ref[...]
    h1 = jnp.maximum(_bnorm(h, g_ref[...], bt_ref[...]), 0.0)
    xl2_ref[...] = (jnp.dot(h1, w_ref[...],
                            preferred_element_type=jnp.float32) + b_ref[...])


def _comb1(p, cinv, g, bt, w, b):
    return pl.pallas_call(
        _comb1_body,
        out_shape=jax.ShapeDtypeStruct((N, D), jnp.float32),
    )(p, cinv, g, bt, w, b)


def _comb2_body(p_ref, cinv_ref, h0_ref, o_ref):
    o_ref[...] = (p_ref[0, :N] + p_ref[1, :N]) * cinv_ref[...] + h0_ref[...]


def _comb2(p, cinv, h0):
    return pl.pallas_call(
        _comb2_body,
        out_shape=jax.ShapeDtypeStruct((N, D), jnp.float32),
    )(p, cinv, h0)


def kernel(x, edge_index, edge_attr,
           Wn0, bn0, W1_0, b1_0, W2_0, b2_0,
           Wn1, bn1, W1_1, b1_1, W2_1, b2_1,
           Wn2, bn2, W1_2, b1_2, W2_2, b2_2,
           gamma0, beta0, gamma1, beta1):
    src = edge_index[0]
    dst = edge_index[1]
    dst3 = dst.reshape(NW, _NCHC, _KC)

    r = lambda v: v.reshape(1, D)
    g_mat = _head_mask()
    cnt = _get_sc_counts()(dst3)
    att0 = _att0(edge_attr, g_mat, (W1_0, r(b1_0), W2_0, r(b2_0)))
    att1, att2, ea_out = _att12(edge_attr, g_mat,
                                (W1_1, r(b1_1), W2_1, r(b2_1),
                                 W1_2, r(b1_2), W2_2, r(b2_2)))

    xl0 = _lin(x, Wn0, r(bn0))
    p0 = _get_sc_agg()(xl0, att0, src, dst)
    h0, xl1, cinv = _comb0(p0, cnt, r(gamma0), r(beta0), Wn1, r(bn1))
    p1 = _get_sc_agg()(xl1, att1, src, dst)
    xl2 = _comb1(p1, cinv, r(gamma1), r(beta1), Wn2, r(bn2))
    p2 = _get_sc_agg()(xl2, att2, src, dst)
    out = _comb2(p2, cinv, h0)
    return (out, ea_out)
